# Initial kernel scaffold; baseline (speedup 1.0000x reference)
import jax
import jax.numpy as jnp
from jax.experimental import pallas as pl


def kernel(boxes, classification):
    raise NotImplementedError("write your pallas kernel here")


# speedup vs baseline: 5.2091x; 5.2091x over previous
import functools

import jax
import jax.numpy as jnp
from jax.experimental import pallas as pl
from jax.experimental.pallas import tpu as pltpu

_NUM_CLASSES = 80
_SCORE_THRESHOLD = 0.05
_IOU_THRESHOLD = 0.5
_MAX_DET = 100

_N = 20000
_B = 8
_CHUNK = 2000
_NCHUNK = _N // _CHUNK
_ROWS = 160
_NPAD = _ROWS * 128


def _score_label_body(cls_ref, sc_ref, lb_ref):
    x = cls_ref[0]
    m = jnp.max(x, axis=1)
    cls_iota = jax.lax.broadcasted_iota(jnp.int32, x.shape, 1)
    lab = jnp.min(jnp.where(x == m[:, None], cls_iota, 2 ** 30),
                  axis=1).astype(jnp.float32)
    sm = jnp.where(m > _SCORE_THRESHOLD, m, -jnp.inf)
    sc_ref[0, 0, 0, :] = sm
    lb_ref[0, 0, 0, :] = lab


def _nms_body(sc_in_ref, bx_ref, lb_ref, out_ref, sc_ref):
    sc_ref[...] = sc_in_ref[0]

    lane_i = jax.lax.broadcasted_iota(jnp.int32, (1, 128), 1)
    row_i = jax.lax.iota(jnp.int32, _ROWS)

    def extract(vec, onehot):
        return jnp.sum(jnp.where(onehot, vec, 0.0))

    rm0 = jnp.max(sc_ref[...], axis=1)
    m0 = jnp.max(rm0)

    zeros = jnp.zeros((1, 128), jnp.float32)
    neg1 = zeros - 1.0
    init = (rm0, m0, neg1, neg1, neg1, neg1, neg1, neg1, zeros,
            jnp.int32(0))

    def cond(st):
        _, m, *_, nk = st
        return jnp.logical_and(nk < _MAX_DET, m > -jnp.inf)

    def body(st):
        rm, m, x1k, y1k, x2k, y2k, sck, lbk, okk, nk = st
        c = jnp.min(jnp.where(rm == m, row_i, jnp.int32(2 ** 30)))
        row = sc_ref[pl.ds(c, 1), :]
        j = jnp.min(jnp.where(row == m, lane_i, jnp.int32(2 ** 30)))
        i = c * 128 + j

        brow = bx_ref[0, pl.ds(i // 32, 1), :]
        l0 = 4 * (i % 32)
        x1 = extract(brow, lane_i == l0)
        y1 = extract(brow, lane_i == l0 + 1)
        x2 = extract(brow, lane_i == l0 + 2)
        y2 = extract(brow, lane_i == l0 + 3)
        lrow = lb_ref[0, pl.ds(c, 1), :]
        lab = extract(lrow, lane_i == j)

        ix1 = jnp.maximum(x1, x1k)
        iy1 = jnp.maximum(y1, y1k)
        ix2 = jnp.minimum(x2, x2k)
        iy2 = jnp.minimum(y2, y2k)
        inter = jnp.clip(ix2 - ix1, 0.0) * jnp.clip(iy2 - iy1, 0.0)
        a1 = (x2 - x1) * (y2 - y1)
        a2 = (x2k - x1k) * (y2k - y1k)
        iou = inter / (a1 + a2 - inter + 1e-8)
        supp = jnp.max(jnp.where(okk > 0.0, iou, -1.0)) > _IOU_THRESHOLD

        slot = lane_i == nk
        keep = jnp.logical_not(supp)

        def upd(old, val):
            return jnp.where(jnp.logical_and(slot, keep), val, old)

        x1k = upd(x1k, x1)
        y1k = upd(y1k, y1)
        x2k = upd(x2k, x2)
        y2k = upd(y2k, y2)
        sck = upd(sck, m)
        lbk = upd(lbk, lab)
        okk = upd(okk, 1.0)
        nk = nk + jnp.where(keep, 1, 0)

        row_new = jnp.where(lane_i == j, -jnp.inf, row)
        sc_ref[pl.ds(c, 1), :] = row_new
        rm = jnp.where(row_i == c, jnp.max(row_new), rm)
        m = jnp.max(rm)
        return (rm, m, x1k, y1k, x2k, y2k, sck, lbk, okk, nk)

    (_, _, x1k, y1k, x2k, y2k, sck, lbk, okk, _) = jax.lax.while_loop(
        cond, body, init)

    out_ref[0, 0:1, :] = x1k
    out_ref[0, 1:2, :] = y1k
    out_ref[0, 2:3, :] = x2k
    out_ref[0, 3:4, :] = y2k
    out_ref[0, 4:5, :] = sck
    out_ref[0, 5:6, :] = lbk
    out_ref[0, 6:7, :] = okk
    out_ref[0, 7:8, :] = zeros


@jax.jit
def kernel(boxes, classification):
    sc4, lb4 = pl.pallas_call(
        _score_label_body,
        grid=(_B, _NCHUNK),
        in_specs=[pl.BlockSpec((1, _CHUNK, _NUM_CLASSES),
                               lambda b, n: (b, n, 0))],
        out_specs=[pl.BlockSpec((1, 1, 1, _CHUNK), lambda b, n: (b, n, 0, 0)),
                   pl.BlockSpec((1, 1, 1, _CHUNK), lambda b, n: (b, n, 0, 0))],
        out_shape=[jax.ShapeDtypeStruct((_B, _NCHUNK, 1, _CHUNK), jnp.float32),
                   jax.ShapeDtypeStruct((_B, _NCHUNK, 1, _CHUNK), jnp.float32)],
        compiler_params=pltpu.CompilerParams(
            dimension_semantics=("parallel", "parallel")),
    )(classification)

    pad = _NPAD - _N
    sc = jnp.pad(sc4.reshape(_B, _N), ((0, 0), (0, pad)),
                 constant_values=-jnp.inf).reshape(_B, _ROWS, 128)
    lb = jnp.pad(lb4.reshape(_B, _N), ((0, 0), (0, pad))
                 ).reshape(_B, _ROWS, 128)
    bx = boxes.reshape(_B, _N * 4 // 128, 128)

    packed = pl.pallas_call(
        _nms_body,
        grid=(_B,),
        in_specs=[pl.BlockSpec((1, _ROWS, 128), lambda b: (b, 0, 0)),
                  pl.BlockSpec((1, _N * 4 // 128, 128), lambda b: (b, 0, 0)),
                  pl.BlockSpec((1, _ROWS, 128), lambda b: (b, 0, 0))],
        out_specs=pl.BlockSpec((1, 8, 128), lambda b: (b, 0, 0)),
        out_shape=jax.ShapeDtypeStruct((_B, 8, 128), jnp.float32),
        scratch_shapes=[pltpu.VMEM((_ROWS, 128), jnp.float32)],
        compiler_params=pltpu.CompilerParams(
            dimension_semantics=("parallel",)),
    )(sc, bx, lb)

    out_boxes = packed[:, 0:4, :_MAX_DET].transpose(0, 2, 1)
    out_scores = packed[:, 4, :_MAX_DET]
    out_labels = packed[:, 5, :_MAX_DET].astype(jnp.int32)
    ok = packed[:, 6, :_MAX_DET] > 0.5
    valid = jnp.sum(ok.astype(jnp.int32), axis=1)
    return out_boxes, out_scores, out_labels, valid


# speedup vs baseline: 32.7239x; 6.2821x over previous
import jax
import jax.numpy as jnp
from jax.experimental import pallas as pl
from jax.experimental.pallas import tpu as pltpu

_NUM_CLASSES = 80
_SCORE_THRESHOLD = 0.05
_IOU_THRESHOLD = 0.5
_MAX_DET = 100

_N = 20000
_B = 8
_CHUNK = _N
_NCHUNK = 1
_ROWS = 160
_NPAD = _ROWS * 128
_BIG = 2 ** 30


def _score_label_body(cls_ref, sc_ref, lb_ref):
    x = cls_ref[0]
    m = jnp.max(x, axis=0)
    cls_iota = jax.lax.broadcasted_iota(jnp.int32, x.shape, 0)
    lab = jnp.min(jnp.where(x == m[None, :], cls_iota, _BIG),
                  axis=0).astype(jnp.float32)
    sm = jnp.where(m > _SCORE_THRESHOLD, m, -jnp.inf)
    sc_ref[0, 0, 0, :] = sm
    lb_ref[0, 0, 0, :] = lab


def _nms_body(sc_in_ref, bx_ref, lb_ref,
              x1o, y1o, x2o, y2o, sco, lbo, oko, sc_ref):
    sc_ref[...] = sc_in_ref[...]

    lane1 = jax.lax.broadcasted_iota(jnp.int32, (1, 128), 1)
    lane2 = jax.lax.broadcasted_iota(jnp.int32, (_B, 128), 1)
    riota2 = jax.lax.broadcasted_iota(jnp.int32, (_B, _ROWS), 1)

    rm0 = jnp.max(sc_ref[...], axis=2)
    m0 = jnp.max(rm0, axis=1, keepdims=True)

    zeros = jnp.zeros((_B, 128), jnp.float32)
    neg1 = zeros - 1.0
    init = (rm0, m0, neg1, neg1, neg1, neg1, neg1, neg1, zeros,
            jnp.zeros((_B, 1), jnp.int32))

    def cond(st):
        _, m, *_, nk = st
        return jnp.any(jnp.logical_and(nk < _MAX_DET, m > -jnp.inf))

    def body(st):
        rm, m, x1k, y1k, x2k, y2k, sck, lbk, okk, nk = st
        a = jnp.logical_and(nk < _MAX_DET, m > -jnp.inf)

        maskedc = jnp.where(rm == m, riota2, _BIG)
        cvec = jnp.min(maskedc, axis=1, keepdims=True)

        rows_l, brows_l, lrows_l = [], [], []
        cs, js = [], []
        for g in range(_B):
            c_g = jnp.minimum(jnp.min(maskedc[g]), _ROWS - 1)
            cs.append(c_g)
            rows_l.append(sc_ref[g, pl.ds(c_g, 1), :])
            lrows_l.append(lb_ref[g, pl.ds(c_g, 1), :])
        rows = jnp.concatenate(rows_l, axis=0)
        maskedj = jnp.where(rows == m, lane2, _BIG)
        jvec = jnp.min(maskedj, axis=1, keepdims=True)
        for g in range(_B):
            j_g = jnp.min(maskedj[g])
            br_g = jnp.clip(cs[g] * 4 + j_g // 32, 0, _N * 4 // 128 - 1)
            js.append(j_g)
            brows_l.append(bx_ref[g, pl.ds(br_g, 1), :])
        brows = jnp.concatenate(brows_l, axis=0)

        l0 = 4 * ((cvec * 128 + jvec) % 32)

        def pick(k):
            return jnp.sum(jnp.where(lane2 == l0 + k, brows, 0.0),
                           axis=1, keepdims=True)

        x1, y1, x2, y2 = pick(0), pick(1), pick(2), pick(3)
        lrows = jnp.concatenate(lrows_l, axis=0)
        lab = jnp.sum(jnp.where(lane2 == jvec, lrows, 0.0),
                      axis=1, keepdims=True)

        ix1 = jnp.maximum(x1, x1k)
        iy1 = jnp.maximum(y1, y1k)
        ix2 = jnp.minimum(x2, x2k)
        iy2 = jnp.minimum(y2, y2k)
        inter = jnp.clip(ix2 - ix1, 0.0) * jnp.clip(iy2 - iy1, 0.0)
        a1 = (x2 - x1) * (y2 - y1)
        a2 = (x2k - x1k) * (y2k - y1k)
        iou = inter / (a1 + a2 - inter + 1e-8)
        supp = jnp.max(jnp.where(okk > 0.0, iou, -1.0),
                       axis=1, keepdims=True) > _IOU_THRESHOLD

        keep = jnp.logical_and(a, jnp.logical_not(supp))
        slot = jnp.logical_and(lane2 == nk, keep)

        x1k = jnp.where(slot, x1, x1k)
        y1k = jnp.where(slot, y1, y1k)
        x2k = jnp.where(slot, x2, x2k)
        y2k = jnp.where(slot, y2, y2k)
        sck = jnp.where(slot, m, sck)
        lbk = jnp.where(slot, lab, lbk)
        okk = jnp.where(slot, 1.0, okk)
        nk = nk + jnp.where(keep, 1, 0)

        rows_new = jnp.where(jnp.logical_and(lane2 == jvec, a),
                             -jnp.inf, rows)
        for g in range(_B):
            sc_ref[g, pl.ds(cs[g], 1), :] = rows_new[g:g + 1]
        rmax_new = jnp.max(rows_new, axis=1, keepdims=True)
        rm = jnp.where(riota2 == cvec, rmax_new, rm)
        m = jnp.max(rm, axis=1, keepdims=True)
        return (rm, m, x1k, y1k, x2k, y2k, sck, lbk, okk, nk)

    (_, _, x1k, y1k, x2k, y2k, sck, lbk, okk, _) = jax.lax.while_loop(
        cond, body, init)

    x1o[...] = x1k
    y1o[...] = y1k
    x2o[...] = x2k
    y2o[...] = y2k
    sco[...] = sck
    lbo[...] = lbk
    oko[...] = okk


@jax.jit
def kernel(boxes, classification):
    cls_t = jnp.transpose(classification, (0, 2, 1))
    sc4, lb4 = pl.pallas_call(
        _score_label_body,
        grid=(_B,),
        in_specs=[pl.BlockSpec((1, _NUM_CLASSES, _CHUNK),
                               lambda b: (b, 0, 0))],
        out_specs=[pl.BlockSpec((1, 1, 1, _CHUNK), lambda b: (b, 0, 0, 0)),
                   pl.BlockSpec((1, 1, 1, _CHUNK), lambda b: (b, 0, 0, 0))],
        out_shape=[jax.ShapeDtypeStruct((_B, 1, 1, _CHUNK), jnp.float32),
                   jax.ShapeDtypeStruct((_B, 1, 1, _CHUNK), jnp.float32)],
        compiler_params=pltpu.CompilerParams(
            dimension_semantics=("parallel",)),
    )(cls_t)

    pad = _NPAD - _N
    sc = jnp.pad(sc4.reshape(_B, _N), ((0, 0), (0, pad)),
                 constant_values=-jnp.inf).reshape(_B, _ROWS, 128)
    lb = jnp.pad(lb4.reshape(_B, _N), ((0, 0), (0, pad))
                 ).reshape(_B, _ROWS, 128)
    bx = boxes.reshape(_B, _N * 4 // 128, 128)

    full = lambda s: pl.BlockSpec(s, lambda: (0,) * len(s))
    outs = pl.pallas_call(
        _nms_body,
        grid=(),
        in_specs=[full((_B, _ROWS, 128)),
                  full((_B, _N * 4 // 128, 128)),
                  full((_B, _ROWS, 128))],
        out_specs=[full((_B, 128))] * 7,
        out_shape=[jax.ShapeDtypeStruct((_B, 128), jnp.float32)] * 7,
        scratch_shapes=[pltpu.VMEM((_B, _ROWS, 128), jnp.float32)],
    )(sc, bx, lb)
    x1o, y1o, x2o, y2o, sco, lbo, oko = outs

    out_boxes = jnp.stack([x1o, y1o, x2o, y2o], axis=-1)[:, :_MAX_DET, :]
    out_scores = sco[:, :_MAX_DET]
    out_labels = lbo[:, :_MAX_DET].astype(jnp.int32)
    valid = jnp.sum((oko[:, :_MAX_DET] > 0.5).astype(jnp.int32), axis=1)
    return out_boxes, out_scores, out_labels, valid
